# Initial kernel scaffold; baseline (speedup 1.0000x reference)
import jax
import jax.numpy as jnp
from jax.experimental import pallas as pl


def kernel(indices):
    raise NotImplementedError("write your pallas kernel here")


# speedup vs baseline: 3.7354x; 3.7354x over previous
import jax
import jax.numpy as jnp
from jax import lax
from jax.experimental import pallas as pl

NUM_CAT = 256
B_BLK = 8


def _onehot_body(idx_ref, out_ref):
    idx = idx_ref[...]
    c = lax.broadcasted_iota(jnp.int32, (B_BLK, NUM_CAT, idx.shape[-1]), 1)
    out_ref[...] = (idx[:, None, :] == c).astype(jnp.float32)


def kernel(indices):
    batch, seq = indices.shape
    grid = batch // B_BLK
    return pl.pallas_call(
        _onehot_body,
        grid=(grid,),
        in_specs=[pl.BlockSpec((B_BLK, seq), lambda i: (i, 0))],
        out_specs=pl.BlockSpec((B_BLK, NUM_CAT, seq), lambda i: (i, 0, 0)),
        out_shape=jax.ShapeDtypeStruct((batch, NUM_CAT, seq), jnp.float32),
    )(indices)


# speedup vs baseline: 4.0830x; 1.0931x over previous
import jax
import jax.numpy as jnp
from jax import lax
from jax.experimental import pallas as pl

NUM_CAT = 256
B_BLK = 32


def _onehot_body(idx_ref, out_ref):
    idx = idx_ref[...]
    c = lax.broadcasted_iota(jnp.int32, (B_BLK, NUM_CAT, idx.shape[-1]), 1)
    out_ref[...] = (idx[:, None, :] == c).astype(jnp.float32)


def kernel(indices):
    batch, seq = indices.shape
    grid = batch // B_BLK
    return pl.pallas_call(
        _onehot_body,
        grid=(grid,),
        in_specs=[pl.BlockSpec((B_BLK, seq), lambda i: (i, 0))],
        out_specs=pl.BlockSpec((B_BLK, NUM_CAT, seq), lambda i: (i, 0, 0)),
        out_shape=jax.ShapeDtypeStruct((batch, NUM_CAT, seq), jnp.float32),
    )(indices)
